# baseline (speedup 1.0000x reference)
import jax
import jax.numpy as jnp
from jax import lax
from jax.experimental import pallas as pl
from jax.experimental.pallas import tpu as pltpu
from jax.experimental.pallas import tpu_sc as plsc

BATCH = 4096
TOKENS = 200
DIM = 2048
ROWS = BATCH * TOKENS

NUM_CORES = 2
NUM_SUBCORES = 16
NUM_WORKERS = NUM_CORES * NUM_SUBCORES
PER_W = ROWS // NUM_WORKERS
NVR = PER_W // 16
POSROWS = (PER_W + TOKENS * 15) // 16 + 4
TPAD = 208
NTG = TPAD // 16


def _sc_body(idx_hbm, table_hbm, out_hbm, idx_v, posbin_v, hist_v, off0_v,
             off_v, rlist_v, stage_v, gsem, wsem):
    wid = lax.axis_index("s") * NUM_CORES + lax.axis_index("c")
    base = wid * PER_W
    lanes = lax.iota(jnp.int32, 16)
    zeros = jnp.broadcast_to(0, (16,))

    pltpu.sync_copy(idx_hbm.at[pl.ds(base, PER_W)], idx_v)

    for k in range(NTG):
        hist_v[pl.ds(k * 16, 16)] = zeros

    def dup_stats(v):
        rank = zeros
        cnt = zeros
        for k in range(16):
            eqk = (v == jnp.broadcast_to(v[k], (16,))).astype(jnp.int32)
            rank = rank + jnp.where(lanes > k, eqk, zeros)
            cnt = cnt + eqk
        return rank, cnt

    @pl.loop(0, NVR)
    def _hist(i):
        v = idx_v[pl.ds(i * 16, 16)]
        _, cnt = dup_stats(v)
        h = plsc.load_gather(hist_v, [v])
        plsc.store_scatter(hist_v, [v], h + cnt)

    def pfx(k, acc):
        h = hist_v[pl.ds(pl.multiple_of(k * 16, 16), 16)]
        hpad = ((h + jnp.broadcast_to(15, (16,))) >> 4) << 4
        c = plsc.cumsum(hpad)
        excl = c - hpad + jnp.broadcast_to(acc, (16,))
        off0_v[pl.ds(pl.multiple_of(k * 16, 16), 16)] = excl
        off_v[pl.ds(pl.multiple_of(k * 16, 16), 16)] = excl
        return acc + jnp.max(c)

    lax.fori_loop(0, NTG, pfx, jnp.int32(0))

    @pl.loop(0, NVR)
    def _place(i):
        v = idx_v[pl.ds(i * 16, 16)]
        rank, cnt = dup_stats(v)
        boff = plsc.load_gather(off_v, [v])
        dest = boff + rank
        pos = jnp.broadcast_to(base + i * 16, (16,)) + lanes
        plsc.store_scatter(posbin_v, [dest >> 4, dest & 15], pos)
        plsc.store_scatter(off_v, [v], boff + cnt)

    def start_gather(s):
        pltpu.async_copy(
            table_hbm.at[rlist_v.at[s]], stage_v.at[s], gsem.at[s]
        )

    def wait_gather(s):
        pltpu.make_async_copy(
            table_hbm.at[rlist_v.at[0]], stage_v.at[s], gsem.at[s]
        ).wait()

    def start_scatter(crow, s):
        pltpu.async_copy(
            stage_v.at[s], out_hbm.at[posbin_v.at[crow]], wsem.at[s]
        )

    def wait_scatter(s):
        pltpu.make_async_copy(
            stage_v.at[s], out_hbm.at[posbin_v.at[0]], wsem.at[s]
        ).wait()

    def group_body(g, carry):
        hv = hist_v[pl.ds(pl.multiple_of(g * 16, 16), 16)]
        ov = off0_v[pl.ds(pl.multiple_of(g * 16, 16), 16)]
        c = [carry[0], carry[1]]
        for l in range(16):
            s = l & 1
            r = g * 16 + l
            n = hv[l]
            o0 = ov[l]
            nch = (n + 15) >> 4
            orow = o0 >> 4

            @pl.loop(0, c[s])
            def _(q):
                wait_scatter(s)

            @pl.when(n > 0)
            def _():
                rem = n & 15

                @pl.when(rem > 0)
                def _():
                    lastrow = orow + nch - 1
                    chunkv = posbin_v[lastrow, pl.ds(0, 16)]
                    firstv = posbin_v[orow, pl.ds(0, 16)]
                    fill = jnp.where(
                        lanes >= jnp.broadcast_to(rem, (16,)),
                        jnp.broadcast_to(firstv[0], (16,)), chunkv)
                    posbin_v[lastrow, pl.ds(0, 16)] = fill

                rlist_v[s, pl.ds(0, 16)] = jnp.broadcast_to(r, (16,))
                start_gather(s)
                wait_gather(s)

                @pl.loop(0, nch)
                def _(cc):
                    start_scatter(orow + cc, s)

            c[s] = nch
        return (c[0], c[1])

    cend = lax.fori_loop(0, NTG, group_body, (jnp.int32(0), jnp.int32(0)))

    for s in range(2):
        @pl.loop(0, cend[s])
        def _(q):
            wait_scatter(s)


@jax.jit
def _sc_gather(idx_flat, table):
    mesh = plsc.VectorSubcoreMesh(
        core_axis_name="c", subcore_axis_name="s",
        num_cores=NUM_CORES, num_subcores=NUM_SUBCORES,
    )
    call = pl.kernel(
        _sc_body,
        out_type=jax.ShapeDtypeStruct((ROWS, DIM), jnp.float32),
        mesh=mesh,
        compiler_params=pltpu.CompilerParams(
            needs_layout_passes=False, use_tc_tiling_on_sc=False),
        scratch_types=[
            pltpu.VMEM((PER_W,), jnp.int32),
            pltpu.VMEM((POSROWS, 16), jnp.int32),
            pltpu.VMEM((TPAD,), jnp.int32),
            pltpu.VMEM((TPAD,), jnp.int32),
            pltpu.VMEM((TPAD,), jnp.int32),
            pltpu.VMEM((2, 16), jnp.int32),
            pltpu.VMEM((2, 16, DIM), jnp.float32),
            pltpu.SemaphoreType.DMA((2,)),
            pltpu.SemaphoreType.DMA((2,)),
        ],
    )
    return call(idx_flat, table)


def kernel(indices, embeddings):
    idx_flat = indices.reshape(ROWS).astype(jnp.int32)
    out = _sc_gather(idx_flat, embeddings)
    return out.reshape(BATCH, TOKENS, DIM)


# speedup vs baseline: 2.4178x; 2.4178x over previous
import jax
import jax.numpy as jnp
from jax import lax
from jax.experimental import pallas as pl
from jax.experimental.pallas import tpu as pltpu
from jax.experimental.pallas import tpu_sc as plsc

BATCH = 4096
TOKENS = 200
DIM = 2048
ROWS = BATCH * TOKENS

NUM_CORES = 2
NUM_SUBCORES = 16
NUM_WORKERS = NUM_CORES * NUM_SUBCORES
PER_W = ROWS // NUM_WORKERS
NVR = PER_W // 16
POSWORDS = PER_W + TOKENS * 15 + 64
POSROWS = (POSWORDS + 127) // 128 + 1
TPAD = 208
NTG = TPAD // 16


def _sc_body(idx_hbm, table_hbm, out_hbm, idx_v, posbin_v, hist_v, off0_v,
             off_v, rlist_v, stage_v, gsem, wsem):
    wid = lax.axis_index("s") * NUM_CORES + lax.axis_index("c")
    base = wid * PER_W
    lanes = lax.iota(jnp.int32, 16)
    zeros = jnp.broadcast_to(0, (16,))

    pltpu.sync_copy(idx_hbm.at[pl.ds(base, PER_W)], idx_v)

    for k in range(NTG):
        hist_v[pl.ds(k * 16, 16)] = zeros

    def dup_stats(v):
        rank = zeros
        cnt = zeros
        for k in range(16):
            eqk = (v == jnp.broadcast_to(v[k], (16,))).astype(jnp.int32)
            rank = rank + jnp.where(lanes > k, eqk, zeros)
            cnt = cnt + eqk
        return rank, cnt

    @pl.loop(0, NVR)
    def _hist(i):
        v = idx_v[pl.ds(i * 16, 16)]
        _, cnt = dup_stats(v)
        h = plsc.load_gather(hist_v, [v])
        plsc.store_scatter(hist_v, [v], h + cnt)

    def pfx(k, acc):
        h = hist_v[pl.ds(pl.multiple_of(k * 16, 16), 16)]
        hpad = ((h + jnp.broadcast_to(15, (16,))) >> 4) << 4
        c = plsc.cumsum(hpad)
        excl = c - hpad + jnp.broadcast_to(acc, (16,))
        off0_v[pl.ds(pl.multiple_of(k * 16, 16), 16)] = excl
        off_v[pl.ds(pl.multiple_of(k * 16, 16), 16)] = excl
        return acc + jnp.max(c)

    lax.fori_loop(0, NTG, pfx, jnp.int32(0))

    @pl.loop(0, NVR)
    def _place(i):
        v = idx_v[pl.ds(i * 16, 16)]
        rank, cnt = dup_stats(v)
        boff = plsc.load_gather(off_v, [v])
        dest = boff + rank
        pos = jnp.broadcast_to(base + i * 16, (16,)) + lanes
        plsc.store_scatter(posbin_v, [dest >> 7, dest & 127], pos)
        plsc.store_scatter(off_v, [v], boff + cnt)

    def start_gather(s):
        pltpu.async_copy(
            table_hbm.at[rlist_v.at[s]], stage_v.at[s], gsem.at[s]
        )

    def wait_gather(s):
        pltpu.make_async_copy(
            table_hbm.at[rlist_v.at[0]], stage_v.at[s], gsem.at[s]
        ).wait()

    def start_scatter(w, s):
        pltpu.async_copy(
            stage_v.at[s],
            out_hbm.at[posbin_v.at[w >> 7, pl.ds(pl.multiple_of(w & 127, 16), 16)]],
            wsem.at[s],
        )

    def wait_scatter(s):
        pltpu.make_async_copy(
            stage_v.at[s], out_hbm.at[posbin_v.at[0, pl.ds(0, 16)]], wsem.at[s]
        ).wait()

    def group_body(g, carry):
        hv = hist_v[pl.ds(pl.multiple_of(g * 16, 16), 16)]
        ov = off0_v[pl.ds(pl.multiple_of(g * 16, 16), 16)]
        c = [carry[0], carry[1]]
        for l in range(16):
            s = l & 1
            r = g * 16 + l
            n = hv[l]
            o0 = ov[l]
            nch = (n + 15) >> 4

            @pl.loop(0, c[s])
            def _(q):
                wait_scatter(s)

            @pl.when(n > 0)
            def _():
                rem = n & 15

                @pl.when(rem > 0)
                def _():
                    wl = o0 + (nch - 1) * 16
                    cl = pl.multiple_of(wl & 127, 16)
                    c0 = pl.multiple_of(o0 & 127, 16)
                    chunkv = posbin_v[wl >> 7, pl.ds(cl, 16)]
                    firstv = posbin_v[o0 >> 7, pl.ds(c0, 16)]
                    fill = jnp.where(
                        lanes >= jnp.broadcast_to(rem, (16,)),
                        jnp.broadcast_to(firstv[0], (16,)), chunkv)
                    posbin_v[wl >> 7, pl.ds(cl, 16)] = fill

                rlist_v[s, pl.ds(0, 16)] = jnp.broadcast_to(r, (16,))
                start_gather(s)
                wait_gather(s)

                @pl.loop(0, nch)
                def _(cc):
                    start_scatter(o0 + cc * 16, s)

            c[s] = nch
        return (c[0], c[1])

    cend = lax.fori_loop(0, NTG, group_body, (jnp.int32(0), jnp.int32(0)))

    for s in range(2):
        @pl.loop(0, cend[s])
        def _(q):
            wait_scatter(s)


@jax.jit
def _sc_gather(idx_flat, table):
    mesh = plsc.VectorSubcoreMesh(
        core_axis_name="c", subcore_axis_name="s",
        num_cores=NUM_CORES, num_subcores=NUM_SUBCORES,
    )
    call = pl.kernel(
        _sc_body,
        out_type=jax.ShapeDtypeStruct((ROWS, DIM), jnp.float32),
        mesh=mesh,
        compiler_params=pltpu.CompilerParams(needs_layout_passes=False),
        scratch_types=[
            pltpu.VMEM((PER_W,), jnp.int32),
            pltpu.VMEM((POSROWS, 128), jnp.int32),
            pltpu.VMEM((TPAD,), jnp.int32),
            pltpu.VMEM((TPAD,), jnp.int32),
            pltpu.VMEM((TPAD,), jnp.int32),
            pltpu.VMEM((2, 16), jnp.int32),
            pltpu.VMEM((2, 16, DIM), jnp.float32),
            pltpu.SemaphoreType.DMA((2,)),
            pltpu.SemaphoreType.DMA((2,)),
        ],
    )
    return call(idx_flat, table)


def kernel(indices, embeddings):
    idx_flat = indices.reshape(ROWS).astype(jnp.int32)
    out = _sc_gather(idx_flat, embeddings)
    return out.reshape(BATCH, TOKENS, DIM)
